# baseline (speedup 1.0000x reference)
import functools

import jax
import jax.numpy as jnp
from jax import lax
from jax.experimental import pallas as pl
from jax.experimental.pallas import tpu as pltpu
from jax.experimental.pallas import tpu_sc as plsc


def _table_body(t_ref, w1_ref, b1_ref, w2_ref, b2_ref, out_ref):
    h = jnp.dot(t_ref[...], w1_ref[...], preferred_element_type=jnp.float32)
    h = jnp.maximum(h + b1_ref[...], 0.0)
    out = jnp.dot(h, w2_ref[...], preferred_element_type=jnp.float32)
    out_ref[...] = out + b2_ref[...]


def _logits_table_tc(emb_table, W1, b1, W2, b2):
    vocab, d = emb_table.shape
    return pl.pallas_call(
        _table_body,
        out_shape=jax.ShapeDtypeStruct((vocab, vocab), jnp.float32),
    )(emb_table, W1, b1.reshape(1, d), W2, b2.reshape(1, vocab))


def _make_sc_gather(n, v, nw, k):
    per_w = n // nw
    n_chunks = per_w // k
    assert per_w % k == 0 and k % 8 == 0
    mesh = plsc.VectorSubcoreMesh(core_axis_name="c", subcore_axis_name="s")

    @functools.partial(
        pl.kernel,
        mesh=mesh,
        compiler_params=pltpu.CompilerParams(use_tc_tiling_on_sc=False),
        out_type=jax.ShapeDtypeStruct((n, v), jnp.float32),
        scratch_types=[
            pltpu.VMEM((per_w,), jnp.int32),
            pltpu.VMEM((k, v), jnp.float32),
            pltpu.VMEM((k, v), jnp.float32),
            pltpu.SemaphoreType.DMA,
            pltpu.SemaphoreType.DMA,
        ],
    )
    def gather_kernel(table_hbm, idx_hbm, out_hbm, idx_v, buf_a, buf_b, sem_a, sem_b):
        wid = lax.axis_index("s") * 2 + lax.axis_index("c")
        base = wid * per_w
        pltpu.sync_copy(idx_hbm.at[pl.ds(base, per_w)], idx_v)
        bufs = (buf_a, buf_b)
        sems = (sem_a, sem_b)
        pltpu.async_copy(table_hbm.at[idx_v.at[pl.ds(0, k)]], bufs[0], sems[0])
        for c in range(n_chunks):
            if c + 1 < n_chunks:
                pltpu.async_copy(
                    table_hbm.at[idx_v.at[pl.ds((c + 1) * k, k)]],
                    bufs[(c + 1) % 2],
                    sems[(c + 1) % 2],
                )
            pltpu.make_async_copy(
                table_hbm.at[idx_v.at[pl.ds(c * k, k)]],
                bufs[c % 2],
                sems[c % 2],
            ).wait()
            pltpu.sync_copy(bufs[c % 2], out_hbm.at[pl.ds(base + c * k, k)])

    return gather_kernel


def kernel(x, emb_table, W1, b1, W2, b2):
    b_rows, l = x.shape
    vocab, d = emb_table.shape
    n = b_rows * l
    nw = 32

    logits_table = _logits_table_tc(emb_table, W1, b1, W2, b2)
    gather = _make_sc_gather(n, vocab, nw, k=64)
    out = gather(logits_table, x.reshape(n))
    return out.reshape(b_rows, l, vocab)


# speedup vs baseline: 1.3723x; 1.3723x over previous
import functools

import jax
import jax.numpy as jnp
from jax import lax
from jax.experimental import pallas as pl
from jax.experimental.pallas import tpu as pltpu
from jax.experimental.pallas import tpu_sc as plsc


def _table_body(t_ref, w1_ref, b1_ref, w2_ref, b2_ref, out_ref):
    h = jnp.dot(t_ref[...], w1_ref[...], preferred_element_type=jnp.float32)
    h = jnp.maximum(h + b1_ref[...], 0.0)
    out = jnp.dot(h, w2_ref[...], preferred_element_type=jnp.float32)
    out_ref[...] = out + b2_ref[...]


def _logits_table_tc(emb_table, W1, b1, W2pad, b2pad):
    vocab, d = emb_table.shape
    vpad = W2pad.shape[1]
    return pl.pallas_call(
        _table_body,
        out_shape=jax.ShapeDtypeStruct((vocab, vpad), jnp.float32),
    )(emb_table, W1, b1.reshape(1, d), W2pad, b2pad.reshape(1, vpad))


def _make_sc_gather(n, vpad, nw, k):
    per_w = n // nw
    n_chunks = per_w // k
    assert per_w % k == 0 and k % 8 == 0
    mesh = plsc.VectorSubcoreMesh(core_axis_name="c", subcore_axis_name="s")

    @functools.partial(
        pl.kernel,
        mesh=mesh,
        out_type=jax.ShapeDtypeStruct((n, vpad), jnp.float32),
        scratch_types=[
            pltpu.VMEM((per_w,), jnp.int32),
            pltpu.VMEM((k, vpad), jnp.float32),
            pltpu.VMEM((k, vpad), jnp.float32),
            pltpu.SemaphoreType.DMA,
            pltpu.SemaphoreType.DMA,
        ],
    )
    def gather_kernel(table_hbm, idx_hbm, out_hbm, idx_v, buf_a, buf_b, sem_a, sem_b):
        wid = lax.axis_index("s") * 2 + lax.axis_index("c")
        base = wid * per_w
        pltpu.sync_copy(idx_hbm.at[pl.ds(base, per_w)], idx_v)
        bufs = (buf_a, buf_b)
        sems = (sem_a, sem_b)
        pltpu.async_copy(table_hbm.at[idx_v.at[pl.ds(0, k)]], bufs[0], sems[0])
        for c in range(n_chunks):
            if c + 1 < n_chunks:
                pltpu.async_copy(
                    table_hbm.at[idx_v.at[pl.ds((c + 1) * k, k)]],
                    bufs[(c + 1) % 2],
                    sems[(c + 1) % 2],
                )
            pltpu.make_async_copy(
                table_hbm.at[idx_v.at[pl.ds(c * k, k)]],
                bufs[c % 2],
                sems[c % 2],
            ).wait()
            pltpu.sync_copy(bufs[c % 2], out_hbm.at[pl.ds(base + c * k, k)])

    return gather_kernel


def kernel(x, emb_table, W1, b1, W2, b2):
    b_rows, l = x.shape
    vocab, d = emb_table.shape
    n = b_rows * l
    vpad = (vocab + 127) // 128 * 128
    nw = 32

    W2pad = jnp.pad(W2, ((0, 0), (0, vpad - vocab)))
    b2pad = jnp.pad(b2, (0, vpad - vocab))
    logits_table = _logits_table_tc(emb_table, W1, b1, W2pad, b2pad)

    gather = _make_sc_gather(n, vpad, nw, k=40)
    out = gather(logits_table, x.reshape(n))
    return out[:, :vocab].reshape(b_rows, l, vocab)


# speedup vs baseline: 2.0006x; 1.4579x over previous
import functools

import jax
import jax.numpy as jnp
from jax import lax
from jax.experimental import pallas as pl
from jax.experimental.pallas import tpu as pltpu
from jax.experimental.pallas import tpu_sc as plsc


def _table_body(t_ref, w1_ref, b1_ref, w2_ref, b2_ref, out_ref):
    h = jnp.dot(t_ref[...], w1_ref[...], preferred_element_type=jnp.float32)
    h = jnp.maximum(h + b1_ref[...], 0.0)
    out = jnp.dot(h, w2_ref[...], preferred_element_type=jnp.float32)
    out_ref[...] = out + b2_ref[...]


def _logits_table_tc(emb_table, W1, b1, W2pad, b2pad):
    vocab, d = emb_table.shape
    vpad = W2pad.shape[1]
    return pl.pallas_call(
        _table_body,
        out_shape=jax.ShapeDtypeStruct((vocab, vpad), jnp.float32),
    )(emb_table, W1, b1.reshape(1, d), W2pad, b2pad.reshape(1, vpad))


def _make_sc_gather(b_rows, l, lpad, vpad, nw):
    rows_per_w = b_rows // nw
    idx_per_w = rows_per_w * lpad
    mesh = plsc.VectorSubcoreMesh(core_axis_name="c", subcore_axis_name="s")

    @functools.partial(
        pl.kernel,
        mesh=mesh,
        out_type=jax.ShapeDtypeStruct((b_rows, l, vpad), jnp.float32),
        scratch_types=[
            pltpu.VMEM((idx_per_w,), jnp.int32),
            pltpu.VMEM((l, vpad), jnp.float32),
            pltpu.VMEM((l, vpad), jnp.float32),
            pltpu.SemaphoreType.DMA,
            pltpu.SemaphoreType.DMA,
        ],
    )
    def gather_kernel(table_hbm, idx_hbm, out_hbm, idx_v, buf_a, buf_b, sem_a, sem_b):
        wid = lax.axis_index("s") * 2 + lax.axis_index("c")
        row0 = wid * rows_per_w
        pltpu.sync_copy(idx_hbm.at[pl.ds(wid * idx_per_w, idx_per_w)], idx_v)
        bufs = (buf_a, buf_b)
        sems = (sem_a, sem_b)
        pltpu.async_copy(table_hbm.at[idx_v.at[pl.ds(0, l)]], bufs[0], sems[0])
        for c in range(rows_per_w):
            if c + 1 < rows_per_w:
                pltpu.async_copy(
                    table_hbm.at[idx_v.at[pl.ds((c + 1) * lpad, l)]],
                    bufs[(c + 1) % 2],
                    sems[(c + 1) % 2],
                )
            pltpu.make_async_copy(
                table_hbm.at[idx_v.at[pl.ds(c * lpad, l)]],
                bufs[c % 2],
                sems[c % 2],
            ).wait()
            pltpu.sync_copy(bufs[c % 2], out_hbm.at[row0 + c])

    return gather_kernel


def kernel(x, emb_table, W1, b1, W2, b2):
    b_rows, l = x.shape
    vocab, d = emb_table.shape
    vpad = (vocab + 127) // 128 * 128
    lpad = (l + 7) // 8 * 8
    nw = 32

    W2pad = jnp.pad(W2, ((0, 0), (0, vpad - vocab)))
    b2pad = jnp.pad(b2, (0, vpad - vocab))
    logits_table = _logits_table_tc(emb_table, W1, b1, W2pad, b2pad)

    xpad = jnp.pad(x, ((0, 0), (0, lpad - l))).reshape(b_rows * lpad)
    gather = _make_sc_gather(b_rows, l, lpad, vpad, nw)
    out = gather(logits_table, xpad)
    return out[:, :, :vocab]
